# Initial kernel scaffold; baseline (speedup 1.0000x reference)
import functools

import jax
import jax.numpy as jnp
from jax import lax
from jax.experimental import pallas as pl
from jax.experimental.pallas import tpu as pltpu
from jax.experimental.pallas import tpu_sc as plsc

N_NODES = 10000
N_PAD = 10240
N_EDGES = 320000
D = 128

NC = 2
NS = 16
NW = NC * NS
EBLK = 80
NBLK_TOTAL = N_EDGES // EBLK
NBLK_TILE = N_EDGES // (NW * EBLK)
ROWS_TILE = N_PAD // NS

_sc_mesh = plsc.VectorSubcoreMesh(core_axis_name="c", subcore_axis_name="s")


@functools.partial(
    pl.kernel,
    out_type=jax.ShapeDtypeStruct((NC, N_PAD), jnp.float32),
    mesh=_sc_mesh,
    scratch_types=[
        pltpu.VMEM((NBLK_TILE, EBLK), jnp.int32),
        pltpu.VMEM((EBLK,), jnp.float32),
        pltpu.VMEM_SHARED((N_PAD,), jnp.float32),
    ],
)
def _deg_kernel(didx_hbm, ones_hbm, zerod_hbm, out_hbm, didx_v, ones_v, deg_sh):
    c = lax.axis_index("c")
    s = lax.axis_index("s")
    wid = c * NS + s
    pltpu.sync_copy(zerod_hbm, deg_sh.at[pl.ds(s * ROWS_TILE, ROWS_TILE)])
    pltpu.sync_copy(ones_hbm, ones_v)
    pltpu.sync_copy(didx_hbm.at[pl.ds(wid * NBLK_TILE, NBLK_TILE)], didx_v)
    plsc.subcore_barrier()

    @pl.loop(0, NBLK_TILE)
    def _(j):
        pltpu.sync_copy(ones_v, deg_sh.at[didx_v.at[j]], add=True)

    plsc.subcore_barrier()
    pltpu.sync_copy(
        deg_sh.at[pl.ds(s * ROWS_TILE, ROWS_TILE)],
        out_hbm.at[c, pl.ds(s * ROWS_TILE, ROWS_TILE)],
    )


@functools.partial(
    pl.kernel,
    out_type=jax.ShapeDtypeStruct((NC, N_PAD, D), jnp.float32),
    mesh=_sc_mesh,
    scratch_types=[
        pltpu.VMEM((NBLK_TILE, EBLK), jnp.int32),
        pltpu.VMEM((NBLK_TILE, EBLK), jnp.int32),
        pltpu.VMEM((EBLK, D), jnp.float32),
        pltpu.VMEM_SHARED((N_PAD, D), jnp.float32),
        pltpu.SemaphoreType.DMA,
    ],
)
def _agg_kernel(g_hbm, sidx_hbm, didx_hbm, zeros_hbm, out_hbm,
                sidx_v, didx_v, rows_v, s_sh, sem):
    c = lax.axis_index("c")
    s = lax.axis_index("s")
    wid = c * NS + s
    pltpu.sync_copy(zeros_hbm, s_sh.at[pl.ds(s * ROWS_TILE, ROWS_TILE)])
    pltpu.sync_copy(sidx_hbm.at[pl.ds(wid * NBLK_TILE, NBLK_TILE)], sidx_v)
    pltpu.sync_copy(didx_hbm.at[pl.ds(wid * NBLK_TILE, NBLK_TILE)], didx_v)
    plsc.subcore_barrier()

    @pl.loop(0, NBLK_TILE)
    def _(j):
        pltpu.async_copy(g_hbm.at[sidx_v.at[j]], rows_v, sem).wait()
        pltpu.sync_copy(rows_v, s_sh.at[didx_v.at[j]], add=True)

    plsc.subcore_barrier()
    pltpu.sync_copy(
        s_sh.at[pl.ds(s * ROWS_TILE, ROWS_TILE)],
        out_hbm.at[c, pl.ds(s * ROWS_TILE, ROWS_TILE)],
    )


RBLK = 1000


def _dinv(deg_ref):
    return lax.rsqrt(deg_ref[0] + deg_ref[1] + 1.0)[:, None]


def _tc_first_body(x_ref, w_ref, deg_ref, o_ref):
    h = jnp.dot(x_ref[...], w_ref[...], preferred_element_type=jnp.float32)
    o_ref[...] = h * _dinv(deg_ref)


def _tc_mid_body(s_ref, g_ref, deg_ref, b_ref, w_ref, o_ref):
    dinv = _dinv(deg_ref)
    t = (s_ref[0] + s_ref[1] + g_ref[...]) * dinv + b_ref[...]
    t = jnp.maximum(t, 0.0)
    o_ref[...] = jnp.dot(t, w_ref[...], preferred_element_type=jnp.float32) * dinv


def _tc_last_body(s_ref, g_ref, deg_ref, b_ref, o_ref):
    o_ref[...] = (s_ref[0] + s_ref[1] + g_ref[...]) * _dinv(deg_ref) + b_ref[...]


_row_spec = pl.BlockSpec((RBLK, D), lambda i: (i, 0))
_deg_spec = pl.BlockSpec((NC, RBLK), lambda i: (0, i))
_part_spec = pl.BlockSpec((NC, RBLK, D), lambda i: (0, i, 0))
_w_spec = pl.BlockSpec((D, D), lambda i: (0, 0))
_b_spec = pl.BlockSpec((1, D), lambda i: (0, 0))
_out_shape = jax.ShapeDtypeStruct((N_NODES, D), jnp.float32)
_grid = (N_NODES // RBLK,)

_tc_first = pl.pallas_call(
    _tc_first_body, grid=_grid, out_shape=_out_shape,
    in_specs=[_row_spec, _w_spec, _deg_spec], out_specs=_row_spec)

_tc_mid = pl.pallas_call(
    _tc_mid_body, grid=_grid, out_shape=_out_shape,
    in_specs=[_part_spec, _row_spec, _deg_spec, _b_spec, _w_spec],
    out_specs=_row_spec)

_tc_last = pl.pallas_call(
    _tc_last_body, grid=_grid, out_shape=_out_shape,
    in_specs=[_part_spec, _row_spec, _deg_spec, _b_spec],
    out_specs=_row_spec)


@jax.jit
def kernel(x, edge_index, W1, b1, W2, b2):
    src = edge_index[0].astype(jnp.int32).reshape(NBLK_TOTAL, EBLK)
    dst = edge_index[1].astype(jnp.int32).reshape(NBLK_TOTAL, EBLK)
    ones = jnp.ones((EBLK,), jnp.float32)
    zerod = jnp.zeros((ROWS_TILE,), jnp.float32)
    zeros2d = jnp.zeros((ROWS_TILE, D), jnp.float32)
    b1r = b1.reshape(1, D)
    b2r = b2.reshape(1, D)

    deg = _deg_kernel(dst, ones, zerod)
    g1 = _tc_first(x, W1, deg)
    s1 = _agg_kernel(g1, src, dst, zeros2d)
    g2 = _tc_mid(s1, g1, deg, b1r, W2)
    s2 = _agg_kernel(g2, src, dst, zeros2d)
    return _tc_last(s2, g2, deg, b2r)


# speedup vs baseline: 19.3343x; 19.3343x over previous
import functools

import jax
import jax.numpy as jnp
from jax import lax
from jax.experimental import pallas as pl
from jax.experimental.pallas import tpu as pltpu
from jax.experimental.pallas import tpu_sc as plsc

N_NODES = 10000
N_PAD = 10240
N_EDGES = 320000
D = 128

NC = 2
NS = 16
NW = NC * NS
EBLK = 80
NBLK_TILE = N_EDGES // (NW * EBLK)
ROWS_TILE = N_PAD // NS

_sc_mesh = plsc.VectorSubcoreMesh(
    core_axis_name="c", subcore_axis_name="s", num_cores=NC, num_subcores=NS)


@functools.partial(
    pl.kernel,
    out_type=jax.ShapeDtypeStruct((NC, N_PAD), jnp.float32),
    mesh=_sc_mesh,
    scratch_types=[
        pltpu.VMEM((NBLK_TILE, EBLK), jnp.int32),
        pltpu.VMEM((EBLK,), jnp.float32),
        pltpu.VMEM_SHARED((N_PAD,), jnp.float32),
    ],
)
def _deg_kernel(didx_hbm, ones_hbm, zerod_hbm, out_hbm, didx_v, ones_v, deg_sh):
    c = lax.axis_index("c")
    s = lax.axis_index("s")
    wid = c * NS + s
    pltpu.sync_copy(zerod_hbm, deg_sh.at[pl.ds(s * ROWS_TILE, ROWS_TILE)])
    pltpu.sync_copy(ones_hbm, ones_v)
    pltpu.sync_copy(didx_hbm.at[wid], didx_v)
    plsc.subcore_barrier()

    @pl.loop(0, NBLK_TILE)
    def _(j):
        pltpu.sync_copy(ones_v, deg_sh.at[didx_v.at[j]], add=True)

    plsc.subcore_barrier()
    pltpu.sync_copy(
        deg_sh.at[pl.ds(s * ROWS_TILE, ROWS_TILE)],
        out_hbm.at[c, pl.ds(s * ROWS_TILE, ROWS_TILE)],
    )


@functools.partial(
    pl.kernel,
    out_type=jax.ShapeDtypeStruct((NC, N_PAD, D), jnp.float32),
    mesh=_sc_mesh,
    scratch_types=[
        pltpu.VMEM((NBLK_TILE, EBLK), jnp.int32),
        pltpu.VMEM((NBLK_TILE, EBLK), jnp.int32),
        pltpu.VMEM((EBLK, D), jnp.float32),
        pltpu.VMEM_SHARED((N_PAD, D), jnp.float32),
        pltpu.SemaphoreType.DMA,
    ],
)
def _agg_kernel(g_hbm, sidx_hbm, didx_hbm, zeros_hbm, out_hbm,
                sidx_v, didx_v, rows_v, s_sh, sem):
    c = lax.axis_index("c")
    s = lax.axis_index("s")
    wid = c * NS + s
    pltpu.sync_copy(zeros_hbm, s_sh.at[pl.ds(s * ROWS_TILE, ROWS_TILE)])
    pltpu.sync_copy(sidx_hbm.at[wid], sidx_v)
    pltpu.sync_copy(didx_hbm.at[wid], didx_v)
    plsc.subcore_barrier()

    @pl.loop(0, NBLK_TILE)
    def _(j):
        pltpu.async_copy(g_hbm.at[sidx_v.at[j]], rows_v, sem).wait()
        pltpu.sync_copy(rows_v, s_sh.at[didx_v.at[j]], add=True)

    plsc.subcore_barrier()
    pltpu.sync_copy(
        s_sh.at[pl.ds(s * ROWS_TILE, ROWS_TILE)],
        out_hbm.at[c, pl.ds(s * ROWS_TILE, ROWS_TILE)],
    )


RBLK = 1000


def _dinv(deg_ref):
    return lax.rsqrt(deg_ref[0] + deg_ref[1] + 1.0)


def _tc_first_body(x_ref, w_ref, deg_ref, o_ref):
    h = jnp.dot(x_ref[...], w_ref[...], preferred_element_type=jnp.float32)
    o_ref[...] = h * _dinv(deg_ref)


def _tc_mid_body(s_ref, g_ref, deg_ref, b_ref, w_ref, o_ref):
    dinv = _dinv(deg_ref)
    t = (s_ref[0] + s_ref[1] + g_ref[...]) * dinv + b_ref[...]
    t = jnp.maximum(t, 0.0)
    o_ref[...] = jnp.dot(t, w_ref[...], preferred_element_type=jnp.float32) * dinv


def _tc_last_body(s_ref, g_ref, deg_ref, b_ref, o_ref):
    o_ref[...] = (s_ref[0] + s_ref[1] + g_ref[...]) * _dinv(deg_ref) + b_ref[...]


_row_spec = pl.BlockSpec((RBLK, D), lambda i: (i, 0))
_deg_spec = pl.BlockSpec((NC, RBLK, 1), lambda i: (0, i, 0))
_part_spec = pl.BlockSpec((NC, RBLK, D), lambda i: (0, i, 0))
_w_spec = pl.BlockSpec((D, D), lambda i: (0, 0))
_b_spec = pl.BlockSpec((1, D), lambda i: (0, 0))
_out_shape = jax.ShapeDtypeStruct((N_NODES, D), jnp.float32)
_grid = (N_NODES // RBLK,)

_tc_first = pl.pallas_call(
    _tc_first_body, grid=_grid, out_shape=_out_shape,
    in_specs=[_row_spec, _w_spec, _deg_spec], out_specs=_row_spec)

_tc_mid = pl.pallas_call(
    _tc_mid_body, grid=_grid, out_shape=_out_shape,
    in_specs=[_part_spec, _row_spec, _deg_spec, _b_spec, _w_spec],
    out_specs=_row_spec)

_tc_last = pl.pallas_call(
    _tc_last_body, grid=_grid, out_shape=_out_shape,
    in_specs=[_part_spec, _row_spec, _deg_spec, _b_spec],
    out_specs=_row_spec)


@jax.jit
def kernel(x, edge_index, W1, b1, W2, b2):
    src = edge_index[0].astype(jnp.int32).reshape(NW, NBLK_TILE, EBLK)
    dst = edge_index[1].astype(jnp.int32).reshape(NW, NBLK_TILE, EBLK)
    ones = jnp.ones((EBLK,), jnp.float32)
    zerod = jnp.zeros((ROWS_TILE,), jnp.float32)
    zeros2d = jnp.zeros((ROWS_TILE, D), jnp.float32)
    b1r = b1.reshape(1, D)
    b2r = b2.reshape(1, D)

    deg = _deg_kernel(dst, ones, zerod)
    deg = deg.reshape(NC, N_PAD, 1)
    g1 = _tc_first(x, W1, deg)
    s1 = _agg_kernel(g1, src, dst, zeros2d)
    g2 = _tc_mid(s1, g1, deg, b1r, W2)
    s2 = _agg_kernel(g2, src, dst, zeros2d)
    return _tc_last(s2, g2, deg, b2r)


# speedup vs baseline: 29.8078x; 1.5417x over previous
import functools

import jax
import jax.numpy as jnp
from jax import lax
from jax.experimental import pallas as pl
from jax.experimental.pallas import tpu as pltpu
from jax.experimental.pallas import tpu_sc as plsc

N_NODES = 10000
N_PAD = 10240
N_EDGES = 320000
D = 128

NC = 2
NS = 16
NW = NC * NS
EBLK = 80
NBLK_TILE = N_EDGES // (NW * EBLK)
ROWS_TILE = N_PAD // NS
NBUF = 2
NIBUF = 4

_sc_mesh = plsc.VectorSubcoreMesh(
    core_axis_name="c", subcore_axis_name="s", num_cores=NC, num_subcores=NS)


@functools.partial(
    pl.kernel,
    out_type=jax.ShapeDtypeStruct((NC, N_PAD), jnp.float32),
    mesh=_sc_mesh,
    scratch_types=[
        pltpu.VMEM((NBLK_TILE, EBLK), jnp.int32),
        pltpu.VMEM((EBLK,), jnp.float32),
        pltpu.VMEM_SHARED((N_PAD,), jnp.float32),
    ],
)
def _deg_kernel(didx_hbm, ones_hbm, zerod_hbm, out_hbm, didx_v, ones_v, deg_sh):
    c = lax.axis_index("c")
    s = lax.axis_index("s")
    wid = c * NS + s
    pltpu.sync_copy(zerod_hbm, deg_sh.at[pl.ds(s * ROWS_TILE, ROWS_TILE)])
    pltpu.sync_copy(ones_hbm, ones_v)
    pltpu.sync_copy(didx_hbm.at[wid], didx_v)
    plsc.subcore_barrier()

    @pl.loop(0, NBLK_TILE)
    def _(j):
        pltpu.sync_copy(ones_v, deg_sh.at[didx_v.at[j]], add=True)

    plsc.subcore_barrier()
    pltpu.sync_copy(
        deg_sh.at[pl.ds(s * ROWS_TILE, ROWS_TILE)],
        out_hbm.at[c, pl.ds(s * ROWS_TILE, ROWS_TILE)],
    )


@functools.partial(
    pl.kernel,
    out_type=jax.ShapeDtypeStruct((NC, N_PAD, D), jnp.float32),
    mesh=_sc_mesh,
    scratch_types=[
        pltpu.VMEM((NIBUF, EBLK), jnp.int32),
        pltpu.VMEM((NIBUF, EBLK), jnp.int32),
        pltpu.VMEM((NBUF, EBLK, D), jnp.float32),
        pltpu.VMEM_SHARED((N_PAD, D), jnp.float32),
        pltpu.SemaphoreType.DMA((NIBUF,)),
        pltpu.SemaphoreType.DMA((NIBUF,)),
        pltpu.SemaphoreType.DMA((NBUF,)),
        pltpu.SemaphoreType.DMA((NBUF,)),
    ],
)
def _agg_kernel(g_hbm, sidx_hbm, didx_hbm, zeros_hbm, out_hbm,
                sidx_v, didx_v, rows_v, s_sh, sisem, disem, gsem, ssem):
    c = lax.axis_index("c")
    s = lax.axis_index("s")
    wid = c * NS + s
    pltpu.sync_copy(zeros_hbm, s_sh.at[pl.ds(s * ROWS_TILE, ROWS_TILE)])

    def idx_start(j):
        bi = jnp.bitwise_and(j, NIBUF - 1)
        pltpu.async_copy(sidx_hbm.at[wid, j], sidx_v.at[bi], sisem.at[bi])
        pltpu.async_copy(didx_hbm.at[wid, j], didx_v.at[bi], disem.at[bi])

    def gather_start(j, slot):
        bi = jnp.bitwise_and(j, NIBUF - 1)
        pltpu.make_async_copy(
            sidx_hbm.at[wid, j], sidx_v.at[bi], sisem.at[bi]).wait()
        pltpu.async_copy(g_hbm.at[sidx_v.at[bi]], rows_v.at[slot],
                         gsem.at[slot])

    idx_start(0)
    idx_start(1)
    gather_start(0, 0)
    plsc.subcore_barrier()

    @pl.loop(0, NBLK_TILE)
    def _(j):
        @pl.when(j + 2 < NBLK_TILE)
        def _():
            idx_start(j + 2)

        @pl.when(j + 1 < NBLK_TILE)
        def _():
            slot = jnp.bitwise_and(j + 1, NBUF - 1)

            @pl.when(j >= 1)
            def _():
                bo = jnp.bitwise_and(j - 1, NIBUF - 1)
                pltpu.make_async_copy(
                    rows_v.at[slot], s_sh.at[didx_v.at[bo]], ssem.at[slot]
                ).wait()

            gather_start(j + 1, slot)

        b = jnp.bitwise_and(j, NBUF - 1)
        bi = jnp.bitwise_and(j, NIBUF - 1)
        pltpu.make_async_copy(
            g_hbm.at[sidx_v.at[bi]], rows_v.at[b], gsem.at[b]).wait()
        pltpu.make_async_copy(
            didx_hbm.at[wid, j], didx_v.at[bi], disem.at[bi]).wait()
        pltpu.async_copy(rows_v.at[b], s_sh.at[didx_v.at[bi]], ssem.at[b],
                         add=True)

    for _j in (NBLK_TILE - 2, NBLK_TILE - 1):
        _b = _j % NBUF
        _bi = _j % NIBUF
        pltpu.make_async_copy(
            rows_v.at[_b], s_sh.at[didx_v.at[_bi]], ssem.at[_b]).wait()

    plsc.subcore_barrier()
    pltpu.sync_copy(
        s_sh.at[pl.ds(s * ROWS_TILE, ROWS_TILE)],
        out_hbm.at[c, pl.ds(s * ROWS_TILE, ROWS_TILE)],
    )


RBLK = 1000


def _dinv(deg_ref):
    return lax.rsqrt(deg_ref[0] + deg_ref[1] + 1.0)


def _tc_first_body(x_ref, w_ref, deg_ref, o_ref):
    h = jnp.dot(x_ref[...], w_ref[...], preferred_element_type=jnp.float32)
    o_ref[...] = h * _dinv(deg_ref)


def _tc_mid_body(s_ref, g_ref, deg_ref, b_ref, w_ref, o_ref):
    dinv = _dinv(deg_ref)
    t = (s_ref[0] + s_ref[1] + g_ref[...]) * dinv + b_ref[...]
    t = jnp.maximum(t, 0.0)
    o_ref[...] = jnp.dot(t, w_ref[...], preferred_element_type=jnp.float32) * dinv


def _tc_last_body(s_ref, g_ref, deg_ref, b_ref, o_ref):
    o_ref[...] = (s_ref[0] + s_ref[1] + g_ref[...]) * _dinv(deg_ref) + b_ref[...]


_row_spec = pl.BlockSpec((RBLK, D), lambda i: (i, 0))
_deg_spec = pl.BlockSpec((NC, RBLK, 1), lambda i: (0, i, 0))
_part_spec = pl.BlockSpec((NC, RBLK, D), lambda i: (0, i, 0))
_w_spec = pl.BlockSpec((D, D), lambda i: (0, 0))
_b_spec = pl.BlockSpec((1, D), lambda i: (0, 0))
_out_shape = jax.ShapeDtypeStruct((N_NODES, D), jnp.float32)
_grid = (N_NODES // RBLK,)

_tc_first = pl.pallas_call(
    _tc_first_body, grid=_grid, out_shape=_out_shape,
    in_specs=[_row_spec, _w_spec, _deg_spec], out_specs=_row_spec)

_tc_mid = pl.pallas_call(
    _tc_mid_body, grid=_grid, out_shape=_out_shape,
    in_specs=[_part_spec, _row_spec, _deg_spec, _b_spec, _w_spec],
    out_specs=_row_spec)

_tc_last = pl.pallas_call(
    _tc_last_body, grid=_grid, out_shape=_out_shape,
    in_specs=[_part_spec, _row_spec, _deg_spec, _b_spec],
    out_specs=_row_spec)


@jax.jit
def kernel(x, edge_index, W1, b1, W2, b2):
    src = edge_index[0].astype(jnp.int32).reshape(NW, NBLK_TILE, EBLK)
    dst = edge_index[1].astype(jnp.int32).reshape(NW, NBLK_TILE, EBLK)
    ones = jnp.ones((EBLK,), jnp.float32)
    zerod = jnp.zeros((ROWS_TILE,), jnp.float32)
    zeros2d = jnp.zeros((ROWS_TILE, D), jnp.float32)
    b1r = b1.reshape(1, D)
    b2r = b2.reshape(1, D)

    deg = _deg_kernel(dst, ones, zerod)
    deg = deg.reshape(NC, N_PAD, 1)
    g1 = _tc_first(x, W1, deg)
    s1 = _agg_kernel(g1, src, dst, zeros2d)
    g2 = _tc_mid(s1, g1, deg, b1r, W2)
    s2 = _agg_kernel(g2, src, dst, zeros2d)
    return _tc_last(s2, g2, deg, b2r)


# speedup vs baseline: 35.3345x; 1.1854x over previous
import functools

import jax
import jax.numpy as jnp
from jax import lax
from jax.experimental import pallas as pl
from jax.experimental.pallas import tpu as pltpu
from jax.experimental.pallas import tpu_sc as plsc

N_NODES = 10000
N_PAD = 10240
N_EDGES = 320000
D = 128

NC = 2
NS = 16
NW = NC * NS
EBLK = 80
NBLK_TILE = N_EDGES // (NW * EBLK)
ROWS_TILE = N_PAD // NS
NBUF = 3
NIBUF = 4

_sc_mesh = plsc.VectorSubcoreMesh(
    core_axis_name="c", subcore_axis_name="s", num_cores=NC, num_subcores=NS)


@functools.partial(
    pl.kernel,
    out_type=jax.ShapeDtypeStruct((NC, N_PAD), jnp.float32),
    mesh=_sc_mesh,
    scratch_types=[
        pltpu.VMEM((NBLK_TILE, EBLK), jnp.int32),
        pltpu.VMEM((EBLK,), jnp.float32),
        pltpu.VMEM_SHARED((N_PAD,), jnp.float32),
        pltpu.SemaphoreType.DMA((NBUF,)),
    ],
)
def _deg_kernel(didx_hbm, ones_hbm, zerod_hbm, out_hbm, didx_v, ones_v,
                deg_sh, ssem):
    c = lax.axis_index("c")
    s = lax.axis_index("s")
    wid = c * NS + s
    pltpu.sync_copy(zerod_hbm, deg_sh.at[pl.ds(s * ROWS_TILE, ROWS_TILE)])
    pltpu.sync_copy(ones_hbm, ones_v)
    pltpu.sync_copy(didx_hbm.at[wid], didx_v)
    plsc.subcore_barrier()

    @pl.loop(0, NBLK_TILE)
    def _(j):
        @pl.when(j >= NBUF)
        def _():
            slot = lax.rem(j, NBUF)
            pltpu.make_async_copy(
                ones_v, deg_sh.at[didx_v.at[j - NBUF]], ssem.at[slot]).wait()

        pltpu.async_copy(ones_v, deg_sh.at[didx_v.at[j]],
                         ssem.at[lax.rem(j, NBUF)], add=True)

    for _j in range(NBLK_TILE - NBUF, NBLK_TILE):
        pltpu.make_async_copy(
            ones_v, deg_sh.at[didx_v.at[_j]], ssem.at[_j % NBUF]).wait()

    plsc.subcore_barrier()
    pltpu.sync_copy(
        deg_sh.at[pl.ds(s * ROWS_TILE, ROWS_TILE)],
        out_hbm.at[c, pl.ds(s * ROWS_TILE, ROWS_TILE)],
    )


@functools.partial(
    pl.kernel,
    out_type=jax.ShapeDtypeStruct((NC, N_PAD, D), jnp.float32),
    mesh=_sc_mesh,
    scratch_types=[
        pltpu.VMEM((NIBUF, EBLK), jnp.int32),
        pltpu.VMEM((NIBUF, EBLK), jnp.int32),
        pltpu.VMEM((NBUF, EBLK, D), jnp.float32),
        pltpu.VMEM_SHARED((N_PAD, D), jnp.float32),
        pltpu.SemaphoreType.DMA((NIBUF,)),
        pltpu.SemaphoreType.DMA((NIBUF,)),
        pltpu.SemaphoreType.DMA((NBUF,)),
        pltpu.SemaphoreType.DMA((NBUF,)),
    ],
)
def _agg_kernel(g_hbm, sidx_hbm, didx_hbm, zeros_hbm, out_hbm,
                sidx_v, didx_v, rows_v, s_sh, sisem, disem, gsem, ssem):
    c = lax.axis_index("c")
    s = lax.axis_index("s")
    wid = c * NS + s
    pltpu.sync_copy(zeros_hbm, s_sh.at[pl.ds(s * ROWS_TILE, ROWS_TILE)])

    def idx_start(j):
        bi = jnp.bitwise_and(j, NIBUF - 1)
        pltpu.async_copy(sidx_hbm.at[wid, j], sidx_v.at[bi], sisem.at[bi])
        pltpu.async_copy(didx_hbm.at[wid, j], didx_v.at[bi], disem.at[bi])

    def gather_start(j, slot):
        bi = jnp.bitwise_and(j, NIBUF - 1)
        pltpu.make_async_copy(
            sidx_hbm.at[wid, j], sidx_v.at[bi], sisem.at[bi]).wait()
        pltpu.async_copy(g_hbm.at[sidx_v.at[bi]], rows_v.at[slot],
                         gsem.at[slot])

    idx_start(0)
    idx_start(1)
    idx_start(2)
    gather_start(0, 0)
    gather_start(1, 1)
    plsc.subcore_barrier()

    @pl.loop(0, NBLK_TILE)
    def _(j):
        @pl.when(j >= 1)
        def _():
            slot = lax.rem(j + 2, NBUF)
            bo = jnp.bitwise_and(j - 1, NIBUF - 1)
            pltpu.make_async_copy(
                rows_v.at[slot], s_sh.at[didx_v.at[bo]], ssem.at[slot]
            ).wait()

        @pl.when(j + 3 < NBLK_TILE)
        def _():
            idx_start(j + 3)

        @pl.when(j + 2 < NBLK_TILE)
        def _():
            gather_start(j + 2, lax.rem(j + 2, NBUF))

        b = lax.rem(j, NBUF)
        bi = jnp.bitwise_and(j, NIBUF - 1)
        pltpu.make_async_copy(
            g_hbm.at[sidx_v.at[bi]], rows_v.at[b], gsem.at[b]).wait()
        pltpu.make_async_copy(
            didx_hbm.at[wid, j], didx_v.at[bi], disem.at[bi]).wait()
        pltpu.async_copy(rows_v.at[b], s_sh.at[didx_v.at[bi]], ssem.at[b],
                         add=True)

    _j = NBLK_TILE - 1
    pltpu.make_async_copy(
        rows_v.at[_j % NBUF], s_sh.at[didx_v.at[_j % NIBUF]],
        ssem.at[_j % NBUF]).wait()

    plsc.subcore_barrier()
    pltpu.sync_copy(
        s_sh.at[pl.ds(s * ROWS_TILE, ROWS_TILE)],
        out_hbm.at[c, pl.ds(s * ROWS_TILE, ROWS_TILE)],
    )


RBLK = 1000


def _dinv(deg_ref):
    return lax.rsqrt(deg_ref[0] + deg_ref[1] + 1.0)


def _tc_first_body(x_ref, w_ref, deg_ref, o_ref):
    h = jnp.dot(x_ref[...], w_ref[...], preferred_element_type=jnp.float32)
    o_ref[...] = h * _dinv(deg_ref)


def _tc_mid_body(s_ref, g_ref, deg_ref, b_ref, w_ref, o_ref):
    dinv = _dinv(deg_ref)
    t = (s_ref[0] + s_ref[1] + g_ref[...]) * dinv + b_ref[...]
    t = jnp.maximum(t, 0.0)
    o_ref[...] = jnp.dot(t, w_ref[...], preferred_element_type=jnp.float32) * dinv


def _tc_last_body(s_ref, g_ref, deg_ref, b_ref, o_ref):
    o_ref[...] = (s_ref[0] + s_ref[1] + g_ref[...]) * _dinv(deg_ref) + b_ref[...]


_row_spec = pl.BlockSpec((RBLK, D), lambda i: (i, 0))
_deg_spec = pl.BlockSpec((NC, RBLK, 1), lambda i: (0, i, 0))
_part_spec = pl.BlockSpec((NC, RBLK, D), lambda i: (0, i, 0))
_w_spec = pl.BlockSpec((D, D), lambda i: (0, 0))
_b_spec = pl.BlockSpec((1, D), lambda i: (0, 0))
_out_shape = jax.ShapeDtypeStruct((N_NODES, D), jnp.float32)
_grid = (N_NODES // RBLK,)

_tc_first = pl.pallas_call(
    _tc_first_body, grid=_grid, out_shape=_out_shape,
    in_specs=[_row_spec, _w_spec, _deg_spec], out_specs=_row_spec)

_tc_mid = pl.pallas_call(
    _tc_mid_body, grid=_grid, out_shape=_out_shape,
    in_specs=[_part_spec, _row_spec, _deg_spec, _b_spec, _w_spec],
    out_specs=_row_spec)

_tc_last = pl.pallas_call(
    _tc_last_body, grid=_grid, out_shape=_out_shape,
    in_specs=[_part_spec, _row_spec, _deg_spec, _b_spec],
    out_specs=_row_spec)


@jax.jit
def kernel(x, edge_index, W1, b1, W2, b2):
    src = edge_index[0].astype(jnp.int32).reshape(NW, NBLK_TILE, EBLK)
    dst = edge_index[1].astype(jnp.int32).reshape(NW, NBLK_TILE, EBLK)
    ones = jnp.ones((EBLK,), jnp.float32)
    zerod = jnp.zeros((ROWS_TILE,), jnp.float32)
    zeros2d = jnp.zeros((ROWS_TILE, D), jnp.float32)
    b1r = b1.reshape(1, D)
    b2r = b2.reshape(1, D)

    deg = _deg_kernel(dst, ones, zerod)
    deg = deg.reshape(NC, N_PAD, 1)
    g1 = _tc_first(x, W1, deg)
    s1 = _agg_kernel(g1, src, dst, zeros2d)
    g2 = _tc_mid(s1, g1, deg, b1r, W2)
    s2 = _agg_kernel(g2, src, dst, zeros2d)
    return _tc_last(s2, g2, deg, b2r)
